# Initial kernel scaffold; baseline (speedup 1.0000x reference)
import jax
import jax.numpy as jnp
from jax.experimental import pallas as pl


def kernel(coordinates, rot, trans, topologies, W_pos, b_pos, W_edge, b_edge, ln_scale, ln_bias):
    raise NotImplementedError("write your pallas kernel here")


# speedup vs baseline: 13.7063x; 13.7063x over previous
import functools

import jax
import jax.numpy as jnp
import ml_dtypes
import numpy as np
from jax import lax
from jax.experimental import pallas as pl
from jax.experimental.pallas import tpu as pltpu
from jax.experimental.pallas import tpu_sc as plsc

NUM_RBF = 16
MAX_REL = 32
F_DIM = 32
ROWS_PER_TILE = 64

_X_IDX = [1, 0, 2, 3, 4, 1, 1, 1, 1, 0, 0, 0, 4, 4, 3, 0, 2, 3, 4, 2, 3, 4, 2, 3, 2]
_Y_IDX = [1, 0, 2, 3, 4, 0, 2, 3, 4, 2, 3, 4, 2, 3, 2, 1, 1, 1, 1, 0, 0, 0, 4, 4, 3]


def _build_selectors():
    px1 = np.zeros((F_DIM, 128), np.float32)
    py1 = np.zeros((F_DIM, 128), np.float32)
    for c in range(25):
        for u in range(3):
            px1[3 * _X_IDX[c] + u, 3 * c + u] = 1.0
            py1[3 * _Y_IDX[c] + u, 3 * c + u] = 1.0
    for b in range(3):
        for a in range(3):
            px1[24 + b, 75 + 3 * b + a] = 1.0
            py1[24 + b, 75 + 3 * b + a] = 1.0

    px2 = np.zeros((F_DIM, 128), np.float32)
    py2 = np.zeros((F_DIM, 128), np.float32)
    for b in range(3):
        for a in range(3):
            for c in range(3):
                px2[15 + 3 * b + a, 9 * b + 3 * a + c] = 1.0
                py2[15 + 3 * b + c, 9 * b + 3 * a + c] = 1.0
            px2[15 + 3 * b + a, 75 + 3 * b + a] = 1.0

    py1[28, 127] = 1.0
    ej = np.zeros((F_DIM, 128), np.float32)
    ej[27, :] = 1.0

    rsp = np.zeros((256, 128), np.float32)
    for c in range(25):
        for u in range(3):
            rsp[3 * c + u, c] = 1.0
    for b in range(3):
        for a in range(3):
            rsp[128 + 75 + 3 * b + a, 25 + a] = 1.0
            for c in range(3):
                rsp[128 + 9 * b + 3 * a + c, 66 + 3 * a + c] = 2.5
    rsp[127, 126] = rsp[127, 127] = 2.5

    sel = np.zeros((128, 448), np.float32)
    for c in range(28):
        for m in range(NUM_RBF):
            sel[c, NUM_RBF * c + m] = 1.0
    sel = np.concatenate([sel, np.zeros((128, 64), np.float32)], axis=1)
    pyall = np.concatenate([py1, py2, ej], axis=1)
    pxall = np.concatenate([px1, px2], axis=1)
    return pxall, pyall, rsp, sel


_PXALL, _PYALL, _RSP, _SEL = _build_selectors()
_MU04 = (0.4 * (2.0 + (np.arange(512) % NUM_RBF) * (40.0 / (NUM_RBF - 1)))
         ).astype(np.float32)
_MU_HI = _MU04.astype(ml_dtypes.bfloat16)
_MU_LO = (_MU04 - _MU_HI.astype(np.float32)).astype(ml_dtypes.bfloat16)
_SEL_A = _SEL.copy()
_SEL_A[126, :] = -_MU_HI.astype(np.float32)
_SEL_A[127, :] = -_MU_LO.astype(np.float32)
_SELAB = np.concatenate([_SEL_A, _SEL], axis=0)
_RSP2 = np.concatenate([_RSP, _RSP], axis=0)
_PYALL2 = np.concatenate([_PYALL, _PYALL], axis=0)


def _table_kernel(co_ref, ro_ref, tr_ref, out_ref, *, n_res):
    co = co_ref[...]
    ro = ro_ref[...]
    tr = tr_ref[...]
    nat = co[:, 0:3]
    ca = co[:, 3:6]
    cc = co[:, 6:9]
    oo = co[:, 9:12]
    bv = ca - nat
    cv = cc - ca
    a0 = bv[:, 1:2] * cv[:, 2:3] - bv[:, 2:3] * cv[:, 1:2]
    a1 = bv[:, 2:3] * cv[:, 0:1] - bv[:, 0:1] * cv[:, 2:3]
    a2 = bv[:, 0:1] * cv[:, 1:2] - bv[:, 1:2] * cv[:, 0:1]
    av = jnp.concatenate([a0, a1, a2], axis=1)
    cb = -0.58273431 * av + 0.56802827 * bv - 0.54067466 * cv + ca
    rows = co.shape[0]
    jcol = (lax.broadcasted_iota(jnp.int32, (rows, 1), 0) % n_res).astype(jnp.float32)
    one = jnp.ones((rows, 1), jnp.float32)
    pad = jnp.zeros((rows, F_DIM - 29), jnp.float32)
    out_ref[...] = jnp.concatenate([nat, ca, cc, oo, cb, ro, tr, jcol, one, pad], axis=1)


def _build_table(co12, ro9, tr3, n_res):
    rows = co12.shape[0]
    return pl.pallas_call(
        functools.partial(_table_kernel, n_res=n_res),
        out_shape=jax.ShapeDtypeStruct((rows, F_DIM), jnp.float32),
    )(co12, ro9, tr3)


def _gather_sc(table, idx):
    n_edges = idx.shape[0]
    info = plsc.get_sparse_core_info()
    nc, ns = info.num_cores, info.num_subcores
    nw = nc * ns
    per_w = n_edges // nw
    chunk = 128
    n_chunks = per_w // chunk
    idx3 = idx.reshape(nw, n_chunks, chunk)
    mesh = plsc.VectorSubcoreMesh(core_axis_name="c", subcore_axis_name="s")

    @functools.partial(
        pl.kernel,
        mesh=mesh,
        out_type=jax.ShapeDtypeStruct((n_edges, F_DIM), jnp.float32),
        scratch_types=[
            pltpu.VMEM((n_chunks, chunk), jnp.int32),
            pltpu.VMEM((chunk, F_DIM), jnp.float32),
            pltpu.VMEM((chunk, F_DIM), jnp.float32),
            pltpu.SemaphoreType.DMA,
            pltpu.SemaphoreType.DMA,
        ],
        compiler_params=pltpu.CompilerParams(use_tc_tiling_on_sc=False),
    )
    def gather(table_hbm, idx_hbm, out_hbm, idx_v, row0_v, row1_v, sem0, sem1):
        wid = lax.axis_index("s") * nc + lax.axis_index("c")
        base = wid * per_w
        pltpu.sync_copy(idx_hbm.at[wid], idx_v)
        bufs = (row0_v, row1_v)
        sems = (sem0, sem1)
        pend = [None, None]
        for c in range(n_chunks):
            pend[c % 2] = pltpu.async_copy(
                table_hbm.at[idx_v.at[c]], bufs[c % 2], sems[c % 2])
            if c > 0:
                pend[(c - 1) % 2].wait()
                pltpu.sync_copy(bufs[(c - 1) % 2],
                                out_hbm.at[pl.ds(base + (c - 1) * chunk, chunk)])
        pend[(n_chunks - 1) % 2].wait()
        pltpu.sync_copy(bufs[(n_chunks - 1) % 2],
                        out_hbm.at[pl.ds(base + (n_chunks - 1) * chunk, chunk)])

    return gather(table, idx3)


def kernel(coordinates, rot, trans, topologies, W_pos, b_pos, W_edge, b_edge, ln_scale, ln_bias):
    B, N, K = topologies.shape
    E_edges = B * N * K
    rows = B * N

    co12 = coordinates.reshape(rows, 12).astype(jnp.float32)
    ro9 = rot.reshape(rows, 9).astype(jnp.float32)
    tr3 = trans.reshape(rows, 3).astype(jnp.float32)

    table = _build_table(co12, ro9, tr3, N)

    topo = topologies.astype(jnp.int32)
    idx = (topo + (jnp.arange(B, dtype=jnp.int32) * N)[:, None, None]).reshape(E_edges)
    g = _gather_sc(table, idx)

    W_edge = W_edge.astype(jnp.float32)
    w_main = jnp.zeros((512, 128), jnp.float32)
    w_main = w_main.at[0:400].set(W_edge[16:416])
    w_main = w_main.at[400:448].set(W_edge[425:473])
    w_aux = jnp.zeros((128, 128), jnp.float32)
    w_aux = w_aux.at[0:66].set(W_pos.astype(jnp.float32) @ W_edge[0:16])
    w_aux = w_aux.at[66:75].set(W_edge[416:425])
    btot = (b_edge.astype(jnp.float32) + b_pos.astype(jnp.float32) @ W_edge[0:16]).reshape(1, 128)
    ctr_m = jnp.eye(128, dtype=jnp.float32) - 1.0 / 128.0
    w_main = w_main @ ctr_m
    w_aux = w_aux @ ctr_m
    btot = btot @ ctr_m

    tile_e = ROWS_PER_TILE * K
    n_tiles = E_edges // tile_e

    consts = dict(n_res=N, k_nb=K)
    edge_call = pl.pallas_call(
        functools.partial(_edge_body, **consts),
        grid=(n_tiles,),
        in_specs=[
            pl.BlockSpec((tile_e, F_DIM), lambda t: (t, 0)),
            pl.BlockSpec((ROWS_PER_TILE, F_DIM), lambda t: (t, 0)),
            pl.BlockSpec((F_DIM, 256), lambda t: (0, 0)),
            pl.BlockSpec((2 * F_DIM, 384), lambda t: (0, 0)),
            pl.BlockSpec((512, 128), lambda t: (0, 0)),
            pl.BlockSpec((256, 512), lambda t: (0, 0)),
            pl.BlockSpec((512, 128), lambda t: (0, 0)),
            pl.BlockSpec((128, 128), lambda t: (0, 0)),
            pl.BlockSpec((1, 128), lambda t: (0, 0)),
            pl.BlockSpec((1, 128), lambda t: (0, 0)),
            pl.BlockSpec((1, 128), lambda t: (0, 0)),
        ],
        out_specs=pl.BlockSpec((tile_e, 128), lambda t: (t, 0)),
        out_shape=jax.ShapeDtypeStruct((E_edges, 128), jnp.float32),
        compiler_params=pltpu.CompilerParams(
            dimension_semantics=("parallel",),
        ),
    )

    bf = jnp.bfloat16
    e_flat = edge_call(
        g, table,
        jnp.asarray(_PXALL, bf), jnp.asarray(_PYALL2, bf),
        jnp.asarray(_RSP2, bf), jnp.asarray(_SELAB, bf),
        w_main.astype(bf), w_aux.astype(bf), btot,
        ln_scale.reshape(1, 128).astype(jnp.float32),
        ln_bias.reshape(1, 128).astype(jnp.float32),
    )
    E_out = e_flat.reshape(B, N, K, 128)
    nodes = jnp.zeros((B, N, 128), dtype=E_out.dtype)
    return E_out, nodes


def _split2(x):
    xh = x.astype(jnp.bfloat16)
    xl = (x - xh.astype(jnp.float32)).astype(jnp.bfloat16)
    return xh, xl


def _bdot(a_bf16, b_bf16):
    return jnp.dot(a_bf16, b_bf16, preferred_element_type=jnp.float32)


def _sel_dot(xh, xl, s_bf16):
    return _bdot(xh, s_bf16) + _bdot(xl, s_bf16)


def _edge_body(g_ref, f_ref, pxall_ref, pyall_ref,
               rsp_ref, sela_ref, wmain_ref, waux_ref, btot_ref,
               lnsc_ref, lnbs_ref, out_ref, *, n_res, k_nb):
    t = pl.program_id(0)
    gg = g_ref[...]
    f8 = f_ref[...]
    rows = gg.shape[0]

    rep_r = lax.broadcasted_iota(jnp.int32, (rows, 128), 0) // k_nb
    rep_c = lax.broadcasted_iota(jnp.int32, (rows, 128), 1)
    rep2 = ((rep_c % ROWS_PER_TILE == rep_r)
            & (rep_c < 2 * ROWS_PER_TILE)).astype(jnp.bfloat16)

    f8h, f8l = _split2(f8)
    ggh, ggl = _split2(gg)
    oall = _sel_dot(f8h, f8l, pxall_ref[...])
    oh, ol = _split2(oall)
    ownall = _bdot(rep2, jnp.concatenate([oh, ol], axis=0))
    own1 = ownall[:, 0:128]
    own2 = ownall[:, 128:256]
    nball = _bdot(jnp.concatenate([ggh, ggl], axis=1), pyall_ref[...])
    nbr1 = nball[:, 0:128]
    nbr2 = nball[:, 128:256]
    jb = nball[:, 256:384]

    dif = nbr1 - own1
    sq = dif * dif
    lane = lax.broadcasted_iota(jnp.int32, (rows, 128), 1)
    pr = own2 * jnp.where(lane < 64, nbr2, dif)

    sqh, sql = _split2(sq)
    prh, prl = _split2(pr)
    cat4 = jnp.concatenate([sqh, prh, sql, prl], axis=1)
    ch = _bdot(cat4, rsp_ref[...])
    dm = jnp.where(lane < 25, jnp.sqrt(ch + 1e-12), ch)
    dm = dm * 0.4

    dmh, dml = _split2(dm)
    ds = _bdot(jnp.concatenate([dmh, dml], axis=1), sela_ref[...])
    phi = jnp.exp(-jnp.square(ds))

    i0 = t * ROWS_PER_TILE
    ib = ((i0 + lax.broadcasted_iota(jnp.int32, (rows, 128), 0) // k_nb)
          % n_res).astype(jnp.float32)
    dpos = jnp.clip(jb - ib + float(MAX_REL), 0.0, 2.0 * MAX_REL)
    oneh = ((jnp.abs(lane.astype(jnp.float32) - dpos) < 0.5)
            & (lane < 66)).astype(jnp.float32)
    rotmask = (lane >= 66) & (lane < 75)
    aux = oneh + jnp.where(rotmask, dm, 0.0)

    ctr = (_bdot(phi.astype(jnp.bfloat16), wmain_ref[...])
           + _bdot(aux.astype(jnp.bfloat16), waux_ref[...]) + btot_ref[...])
    var = jnp.mean(ctr * ctr, axis=1, keepdims=True)
    out_ref[...] = (ctr * lax.rsqrt(var + 1e-6) * lnsc_ref[...]
                    + lnbs_ref[...])


# speedup vs baseline: 14.1357x; 1.0313x over previous
import functools

import jax
import jax.numpy as jnp
import ml_dtypes
import numpy as np
from jax import lax
from jax.experimental import pallas as pl
from jax.experimental.pallas import tpu as pltpu
from jax.experimental.pallas import tpu_sc as plsc

NUM_RBF = 16
MAX_REL = 32
F_DIM = 32
ROWS_PER_TILE = 64

_X_IDX = [1, 0, 2, 3, 4, 1, 1, 1, 1, 0, 0, 0, 4, 4, 3, 0, 2, 3, 4, 2, 3, 4, 2, 3, 2]
_Y_IDX = [1, 0, 2, 3, 4, 0, 2, 3, 4, 2, 3, 4, 2, 3, 2, 1, 1, 1, 1, 0, 0, 0, 4, 4, 3]


def _build_selectors():
    px1 = np.zeros((F_DIM, 128), np.float32)
    py1 = np.zeros((F_DIM, 128), np.float32)
    for c in range(25):
        for u in range(3):
            px1[3 * _X_IDX[c] + u, 3 * c + u] = 1.0
            py1[3 * _Y_IDX[c] + u, 3 * c + u] = 1.0
    for b in range(3):
        for a in range(3):
            px1[24 + b, 75 + 3 * b + a] = 1.0
            py1[24 + b, 75 + 3 * b + a] = 1.0

    px2 = np.zeros((F_DIM, 128), np.float32)
    for b in range(3):
        for a in range(3):
            for c in range(3):
                px2[15 + 3 * b + a, 84 + 9 * b + 3 * a + c] = 1.0
                py1[15 + 3 * b + c, 84 + 9 * b + 3 * a + c] = 1.0
            px2[15 + 3 * b + a, 75 + 3 * b + a] = 1.0

    py1[28, 127] = 1.0
    ej = np.zeros((F_DIM, 128), np.float32)
    ej[27, :] = 1.0

    rsp = np.zeros((256, 128), np.float32)
    for c in range(25):
        for u in range(3):
            rsp[3 * c + u, c] = 1.0
    for b in range(3):
        for a in range(3):
            rsp[128 + 75 + 3 * b + a, 25 + a] = 1.0
            for c in range(3):
                rsp[128 + 84 + 9 * b + 3 * a + c, 66 + 3 * a + c] = 2.5
    rsp[127, 126] = rsp[127, 127] = 2.5

    sel = np.zeros((128, 448), np.float32)
    for c in range(28):
        for m in range(NUM_RBF):
            sel[c, NUM_RBF * c + m] = 1.0
    sel = np.concatenate([sel, np.zeros((128, 64), np.float32)], axis=1)
    pyall = np.concatenate([py1, ej], axis=1)
    pxall = np.concatenate([px1, px2], axis=1)
    return pxall, pyall, rsp, sel


_PXALL, _PYALL, _RSP, _SEL = _build_selectors()
_MU04 = (0.4 * (2.0 + (np.arange(512) % NUM_RBF) * (40.0 / (NUM_RBF - 1)))
         ).astype(np.float32)
_MU_HI = _MU04.astype(ml_dtypes.bfloat16)
_MU_LO = (_MU04 - _MU_HI.astype(np.float32)).astype(ml_dtypes.bfloat16)
_SEL_A = _SEL.copy()
_SEL_A[126, :] = -_MU_HI.astype(np.float32)
_SEL_A[127, :] = -_MU_LO.astype(np.float32)
_SELAB = np.concatenate([_SEL_A, _SEL], axis=0)
_RSP2 = np.concatenate([_RSP, _RSP], axis=0)
_PYALL2 = np.concatenate([_PYALL, _PYALL], axis=0)


def _table_kernel(co_ref, ro_ref, tr_ref, out_ref, *, n_res):
    co = co_ref[...]
    ro = ro_ref[...]
    tr = tr_ref[...]
    nat = co[:, 0:3]
    ca = co[:, 3:6]
    cc = co[:, 6:9]
    oo = co[:, 9:12]
    bv = ca - nat
    cv = cc - ca
    a0 = bv[:, 1:2] * cv[:, 2:3] - bv[:, 2:3] * cv[:, 1:2]
    a1 = bv[:, 2:3] * cv[:, 0:1] - bv[:, 0:1] * cv[:, 2:3]
    a2 = bv[:, 0:1] * cv[:, 1:2] - bv[:, 1:2] * cv[:, 0:1]
    av = jnp.concatenate([a0, a1, a2], axis=1)
    cb = -0.58273431 * av + 0.56802827 * bv - 0.54067466 * cv + ca
    rows = co.shape[0]
    jcol = (lax.broadcasted_iota(jnp.int32, (rows, 1), 0) % n_res).astype(jnp.float32)
    one = jnp.ones((rows, 1), jnp.float32)
    pad = jnp.zeros((rows, F_DIM - 29), jnp.float32)
    out_ref[...] = jnp.concatenate([nat, ca, cc, oo, cb, ro, tr, jcol, one, pad], axis=1)


def _build_table(co12, ro9, tr3, n_res):
    rows = co12.shape[0]
    return pl.pallas_call(
        functools.partial(_table_kernel, n_res=n_res),
        out_shape=jax.ShapeDtypeStruct((rows, F_DIM), jnp.float32),
    )(co12, ro9, tr3)


def _gather_sc(table, idx):
    n_edges = idx.shape[0]
    info = plsc.get_sparse_core_info()
    nc, ns = info.num_cores, info.num_subcores
    nw = nc * ns
    per_w = n_edges // nw
    chunk = 128
    n_chunks = per_w // chunk
    idx3 = idx.reshape(nw, n_chunks, chunk)
    mesh = plsc.VectorSubcoreMesh(core_axis_name="c", subcore_axis_name="s")

    @functools.partial(
        pl.kernel,
        mesh=mesh,
        out_type=jax.ShapeDtypeStruct((n_edges, F_DIM), jnp.float32),
        scratch_types=[
            pltpu.VMEM((n_chunks, chunk), jnp.int32),
            pltpu.VMEM((chunk, F_DIM), jnp.float32),
            pltpu.VMEM((chunk, F_DIM), jnp.float32),
            pltpu.SemaphoreType.DMA,
            pltpu.SemaphoreType.DMA,
        ],
        compiler_params=pltpu.CompilerParams(use_tc_tiling_on_sc=False),
    )
    def gather(table_hbm, idx_hbm, out_hbm, idx_v, row0_v, row1_v, sem0, sem1):
        wid = lax.axis_index("s") * nc + lax.axis_index("c")
        base = wid * per_w
        pltpu.sync_copy(idx_hbm.at[wid], idx_v)
        bufs = (row0_v, row1_v)
        sems = (sem0, sem1)
        pend = [None, None]
        for c in range(n_chunks):
            pend[c % 2] = pltpu.async_copy(
                table_hbm.at[idx_v.at[c]], bufs[c % 2], sems[c % 2])
            if c > 0:
                pend[(c - 1) % 2].wait()
                pltpu.sync_copy(bufs[(c - 1) % 2],
                                out_hbm.at[pl.ds(base + (c - 1) * chunk, chunk)])
        pend[(n_chunks - 1) % 2].wait()
        pltpu.sync_copy(bufs[(n_chunks - 1) % 2],
                        out_hbm.at[pl.ds(base + (n_chunks - 1) * chunk, chunk)])

    return gather(table, idx3)


def kernel(coordinates, rot, trans, topologies, W_pos, b_pos, W_edge, b_edge, ln_scale, ln_bias):
    B, N, K = topologies.shape
    E_edges = B * N * K
    rows = B * N

    co12 = coordinates.reshape(rows, 12).astype(jnp.float32)
    ro9 = rot.reshape(rows, 9).astype(jnp.float32)
    tr3 = trans.reshape(rows, 3).astype(jnp.float32)

    table = _build_table(co12, ro9, tr3, N)

    topo = topologies.astype(jnp.int32)
    idx = (topo + (jnp.arange(B, dtype=jnp.int32) * N)[:, None, None]).reshape(E_edges)
    g = _gather_sc(table, idx)

    W_edge = W_edge.astype(jnp.float32)
    w_main = jnp.zeros((512, 128), jnp.float32)
    w_main = w_main.at[0:400].set(W_edge[16:416])
    w_main = w_main.at[400:448].set(W_edge[425:473])
    w_aux = jnp.zeros((128, 128), jnp.float32)
    w_aux = w_aux.at[0:66].set(W_pos.astype(jnp.float32) @ W_edge[0:16])
    w_aux = w_aux.at[66:75].set(W_edge[416:425])
    btot = (b_edge.astype(jnp.float32) + b_pos.astype(jnp.float32) @ W_edge[0:16]).reshape(1, 128)
    ctr_m = jnp.eye(128, dtype=jnp.float32) - 1.0 / 128.0
    w_main = w_main @ ctr_m
    w_aux = w_aux @ ctr_m
    btot = btot @ ctr_m

    tile_e = ROWS_PER_TILE * K
    n_tiles = E_edges // tile_e

    consts = dict(n_res=N, k_nb=K)
    edge_call = pl.pallas_call(
        functools.partial(_edge_body, **consts),
        grid=(n_tiles,),
        in_specs=[
            pl.BlockSpec((tile_e, F_DIM), lambda t: (t, 0)),
            pl.BlockSpec((ROWS_PER_TILE, F_DIM), lambda t: (t, 0)),
            pl.BlockSpec((F_DIM, 256), lambda t: (0, 0)),
            pl.BlockSpec((2 * F_DIM, 256), lambda t: (0, 0)),
            pl.BlockSpec((512, 128), lambda t: (0, 0)),
            pl.BlockSpec((256, 512), lambda t: (0, 0)),
            pl.BlockSpec((512, 128), lambda t: (0, 0)),
            pl.BlockSpec((128, 128), lambda t: (0, 0)),
            pl.BlockSpec((1, 128), lambda t: (0, 0)),
            pl.BlockSpec((1, 128), lambda t: (0, 0)),
            pl.BlockSpec((1, 128), lambda t: (0, 0)),
        ],
        out_specs=pl.BlockSpec((tile_e, 128), lambda t: (t, 0)),
        out_shape=jax.ShapeDtypeStruct((E_edges, 128), jnp.float32),
        compiler_params=pltpu.CompilerParams(
            dimension_semantics=("parallel",),
        ),
    )

    bf = jnp.bfloat16
    e_flat = edge_call(
        g, table,
        jnp.asarray(_PXALL, bf), jnp.asarray(_PYALL2, bf),
        jnp.asarray(_RSP2, bf), jnp.asarray(_SELAB, bf),
        w_main.astype(bf), w_aux.astype(bf), btot,
        ln_scale.reshape(1, 128).astype(jnp.float32),
        ln_bias.reshape(1, 128).astype(jnp.float32),
    )
    E_out = e_flat.reshape(B, N, K, 128)
    nodes = jnp.zeros((B, N, 128), dtype=E_out.dtype)
    return E_out, nodes


def _split2(x):
    xh = x.astype(jnp.bfloat16)
    xl = (x - xh.astype(jnp.float32)).astype(jnp.bfloat16)
    return xh, xl


def _bdot(a_bf16, b_bf16):
    return jnp.dot(a_bf16, b_bf16, preferred_element_type=jnp.float32)


def _sel_dot(xh, xl, s_bf16):
    return _bdot(xh, s_bf16) + _bdot(xl, s_bf16)


def _edge_body(g_ref, f_ref, pxall_ref, pyall_ref,
               rsp_ref, sela_ref, wmain_ref, waux_ref, btot_ref,
               lnsc_ref, lnbs_ref, out_ref, *, n_res, k_nb):
    t = pl.program_id(0)
    gg = g_ref[...]
    f8 = f_ref[...]
    rows = gg.shape[0]

    rep_r = lax.broadcasted_iota(jnp.int32, (rows, 128), 0) // k_nb
    rep_c = lax.broadcasted_iota(jnp.int32, (rows, 128), 1)
    rep2 = ((rep_c % ROWS_PER_TILE == rep_r)
            & (rep_c < 2 * ROWS_PER_TILE)).astype(jnp.bfloat16)

    f8h, f8l = _split2(f8)
    ggh, ggl = _split2(gg)
    oall = _sel_dot(f8h, f8l, pxall_ref[...])
    oh, ol = _split2(oall)
    ownall = _bdot(rep2, jnp.concatenate([oh, ol], axis=0))
    own1 = ownall[:, 0:128]
    own2 = ownall[:, 128:256]
    nball = _bdot(jnp.concatenate([ggh, ggl], axis=1), pyall_ref[...])
    nbr1 = nball[:, 0:128]
    jb = nball[:, 128:256]

    dif = nbr1 - own1
    sq = dif * dif
    lane = lax.broadcasted_iota(jnp.int32, (rows, 128), 1)
    pr = own2 * dif

    sqh, sql = _split2(sq)
    prh, prl = _split2(pr)
    cat4 = jnp.concatenate([sqh, prh, sql, prl], axis=1)
    ch = _bdot(cat4, rsp_ref[...])
    dm = jnp.where(lane < 25, jnp.sqrt(ch + 1e-12), ch)
    dm = dm * 0.4

    dmh, dml = _split2(dm)
    ds = _bdot(jnp.concatenate([dmh, dml], axis=1), sela_ref[...])
    phi = jnp.exp(-jnp.square(ds))

    i0 = lax.rem(t * ROWS_PER_TILE, n_res)
    ib = (i0 + lax.broadcasted_iota(jnp.int32, (rows, 128), 0) // k_nb
          ).astype(jnp.float32)
    dpos = jnp.clip(jb - ib + float(MAX_REL), 0.0, 2.0 * MAX_REL)
    oneh = (jnp.abs(lane.astype(jnp.float32) - dpos) < 0.5).astype(jnp.float32)
    rotmask = (lane >= 66) & (lane < 75)
    aux = oneh + jnp.where(rotmask, dm, 0.0)

    ctr = (_bdot(phi.astype(jnp.bfloat16), wmain_ref[...])
           + _bdot(aux.astype(jnp.bfloat16), waux_ref[...]) + btot_ref[...])
    var = jnp.mean(ctr * ctr, axis=1, keepdims=True)
    out_ref[...] = (ctr * lax.rsqrt(var + 1e-6) * lnsc_ref[...]
                    + lnbs_ref[...])
